# baseline (speedup 1.0000x reference)
import functools

import jax
import jax.numpy as jnp
from jax import lax
from jax.experimental import pallas as pl
from jax.experimental.pallas import tpu as pltpu
from jax.experimental.pallas import tpu_sc as plsc

THRESH = 0.5
MARGIN = 1.0
WEIGHT = 1.0
PULL_W = 1.0
PUSH_W = 1.0

_NC = 2
_NS = 16
_NW = _NC * _NS
_L = 16

_B_SC = 32
_CHUNK = 16384
_G_TC = 16


def _sc_body(adj_hbm, pull_hbm, push_hbm, out_hbm,
             a0, a1, p0, p1, q0, q1, sem0, sem1):
    g = lax.axis_index("s") * _NC + lax.axis_index("c")
    nn = adj_hbm.shape[1]
    nchunk = nn // _CHUNK
    abuf, pbuf, qbuf = (a0, a1), (p0, p1), (q0, q1)
    sems = (sem0, sem1)

    def start(c, par):
        base = c * _CHUNK
        return (
            pltpu.async_copy(adj_hbm.at[g, pl.ds(base, _CHUNK)], abuf[par], sems[par]),
            pltpu.async_copy(pull_hbm.at[g, pl.ds(base, _CHUNK)], pbuf[par], sems[par]),
            pltpu.async_copy(push_hbm.at[g, pl.ds(base, _CHUNK)], qbuf[par], sems[par]),
        )

    pull_acc = jnp.zeros((_L,), jnp.float32)
    push_acc = jnp.zeros((_L,), jnp.float32)
    pending = start(0, 0)
    for c in range(nchunk):
        par = c & 1
        nxt = start(c + 1, 1 - par) if c + 1 < nchunk else None
        for h in pending:
            h.wait()
        av, pv, qv = abuf[par], pbuf[par], qbuf[par]

        def inner(i, carry, av=av, pv=pv, qv=qv):
            pa, sa = carry
            off = i * _L
            a = jnp.where(av[pl.ds(off, _L)] > THRESH, 1.0, 0.0)
            dp = a - pv[pl.ds(off, _L)]
            dq = a - qv[pl.ds(off, _L)]
            return pa + dp * dp, sa + dq * dq

        pull_acc, push_acc = lax.fori_loop(
            0, _CHUNK // _L, inner, (pull_acc, push_acc), unroll=4)
        pending = nxt

    pull_sum, push_sum = 0.0, 0.0
    for i in range(_L):
        pull_sum += pull_acc[i]
        push_sum += push_acc[i]
    inv = 1.0 / nn
    pull_mse = pull_sum * inv
    push_mse = push_sum * inv
    contrib = PULL_W * pull_mse + PUSH_W * jnp.maximum(MARGIN - push_mse, 0.0)
    a0[pl.ds(0, _L)] = jnp.full((_L,), contrib, jnp.float32)
    pltpu.sync_copy(a0.at[pl.ds(0, _L)], out_hbm.at[pl.ds(g * _L, _L)])


def _sc_contribs(adj_flat, pull_flat, push_flat):
    nn = adj_flat.shape[1]
    mesh = plsc.VectorSubcoreMesh(
        core_axis_name="c", subcore_axis_name="s",
        num_cores=_NC, num_subcores=_NS)
    vbuf = pltpu.VMEM((_CHUNK,), jnp.float32)
    return pl.kernel(
        _sc_body,
        out_type=jax.ShapeDtypeStruct((_NW * _L,), jnp.float32),
        mesh=mesh,
        scratch_types=[vbuf, vbuf, vbuf, vbuf, vbuf, vbuf,
                       pltpu.SemaphoreType.DMA, pltpu.SemaphoreType.DMA],
    )(adj_flat, pull_flat, push_flat)


def _tc_body(adj_ref, pull_ref, push_ref, out_ref, acc_ref, *, g, n, nsteps):
    i = pl.program_id(0)
    inv_nn = 1.0 / (n * n)

    @pl.when(i == 0)
    def _init():
        acc_ref[0] = 0.0

    contrib = 0.0
    for gg in range(g):
        a = (adj_ref[gg] > THRESH).astype(jnp.float32)
        dp = a - pull_ref[0, gg]
        ds = a - push_ref[0, gg]
        pull_mse = jnp.sum(dp * dp) * inv_nn
        push_mse = jnp.sum(ds * ds) * inv_nn
        contrib += PULL_W * pull_mse + PUSH_W * jnp.maximum(MARGIN - push_mse, 0.0)
    acc_ref[0] += contrib

    @pl.when(i == nsteps - 1)
    def _fin():
        out_ref[0, 0] = acc_ref[0]


def _tc_sum(adj, ref_pull, ref_push):
    B, N, _ = adj.shape
    b_tc = B - _B_SC
    off = _B_SC // _G_TC
    nsteps = b_tc // _G_TC
    out = pl.pallas_call(
        functools.partial(_tc_body, g=_G_TC, n=N, nsteps=nsteps),
        grid=(nsteps,),
        in_specs=[
            pl.BlockSpec((_G_TC, N, N), lambda i: (i + off, 0, 0)),
            pl.BlockSpec((1, _G_TC, N, N), lambda i: (0, i + off, 0, 0)),
            pl.BlockSpec((1, _G_TC, N, N), lambda i: (0, i + off, 0, 0)),
        ],
        out_specs=pl.BlockSpec((1, 1), lambda i: (0, 0), memory_space=pltpu.SMEM),
        out_shape=jax.ShapeDtypeStruct((1, 1), jnp.float32),
        scratch_shapes=[pltpu.SMEM((1,), jnp.float32)],
    )(adj, ref_pull, ref_push)
    return out[0, 0]


def kernel(adj, ref_pull, ref_push):
    B, N, _ = adj.shape
    nn = N * N
    sc = _sc_contribs(
        adj.reshape(B, nn),
        ref_pull.reshape(B, nn),
        ref_push.reshape(B, nn),
    )
    tc = _tc_sum(adj, ref_pull, ref_push)
    return (tc + jnp.sum(sc.reshape(_NW, _L)[:, 0])) * (WEIGHT / B)


# speedup vs baseline: 2.6081x; 2.6081x over previous
import functools

import jax
import jax.numpy as jnp
from jax import lax
from jax.experimental import pallas as pl
from jax.experimental.pallas import tpu as pltpu
from jax.experimental.pallas import tpu_sc as plsc

THRESH = 0.5
MARGIN = 1.0
WEIGHT = 1.0
PULL_W = 1.0
PUSH_W = 1.0

_NC = 2
_NS = 16
_NW = _NC * _NS
_L = 16

_B_SC = 32
_ROWS = 64
_G_TC = 16


def _sc_body(adj_hbm, pull_hbm, push_hbm, out_hbm,
             a0, a1, p0, p1, q0, q1, sem0, sem1):
    g = lax.axis_index("s") * _NC + lax.axis_index("c")
    n = adj_hbm.shape[1]
    nn = n * n
    nchunk = n // _ROWS
    abuf, pbuf, qbuf = (a0, a1), (p0, p1), (q0, q1)
    sems = (sem0, sem1)

    def start(c, par):
        r0 = c * _ROWS
        return (
            pltpu.async_copy(adj_hbm.at[g, pl.ds(r0, _ROWS), :], abuf[par], sems[par]),
            pltpu.async_copy(pull_hbm.at[0, g, pl.ds(r0, _ROWS), :], pbuf[par], sems[par]),
            pltpu.async_copy(push_hbm.at[0, g, pl.ds(r0, _ROWS), :], qbuf[par], sems[par]),
        )

    pull_acc = jnp.zeros((_L,), jnp.float32)
    push_acc = jnp.zeros((_L,), jnp.float32)
    pending = start(0, 0)
    for c in range(nchunk):
        par = c & 1
        nxt = start(c + 1, 1 - par) if c + 1 < nchunk else None
        for h in pending:
            h.wait()
        av, pv, qv = abuf[par], pbuf[par], qbuf[par]

        def inner(r, carry, av=av, pv=pv, qv=qv):
            pa, sa = carry
            for j in range(n // _L):
                sl = pl.ds(j * _L, _L)
                a = jnp.where(av[r, sl] > THRESH, 1.0, 0.0)
                dp = a - pv[r, sl]
                dq = a - qv[r, sl]
                pa = pa + dp * dp
                sa = sa + dq * dq
            return pa, sa

        pull_acc, push_acc = lax.fori_loop(
            0, _ROWS, inner, (pull_acc, push_acc))
        pending = nxt

    pull_sum, push_sum = 0.0, 0.0
    for i in range(_L):
        pull_sum += pull_acc[i]
        push_sum += push_acc[i]
    inv = 1.0 / nn
    pull_mse = pull_sum * inv
    push_mse = push_sum * inv
    contrib = PULL_W * pull_mse + PUSH_W * jnp.maximum(MARGIN - push_mse, 0.0)
    a0[0, pl.ds(0, _L)] = jnp.full((_L,), contrib, jnp.float32)
    pltpu.sync_copy(a0.at[0, pl.ds(0, _L)], out_hbm.at[pl.ds(g * _L, _L)])


def _sc_contribs(adj, ref_pull, ref_push):
    n = adj.shape[1]
    mesh = plsc.VectorSubcoreMesh(
        core_axis_name="c", subcore_axis_name="s",
        num_cores=_NC, num_subcores=_NS)
    vbuf = pltpu.VMEM((_ROWS, n), jnp.float32)
    return pl.kernel(
        _sc_body,
        out_type=jax.ShapeDtypeStruct((_NW * _L,), jnp.float32),
        mesh=mesh,
        scratch_types=[vbuf, vbuf, vbuf, vbuf, vbuf, vbuf,
                       pltpu.SemaphoreType.DMA, pltpu.SemaphoreType.DMA],
    )(adj, ref_pull, ref_push)


def _tc_body(adj_ref, pull_ref, push_ref, out_ref, acc_ref, *, g, n, nsteps):
    i = pl.program_id(0)
    inv_nn = 1.0 / (n * n)

    @pl.when(i == 0)
    def _init():
        acc_ref[0] = 0.0

    contrib = 0.0
    for gg in range(g):
        a = (adj_ref[gg] > THRESH).astype(jnp.float32)
        dp = a - pull_ref[0, gg]
        ds = a - push_ref[0, gg]
        pull_mse = jnp.sum(dp * dp) * inv_nn
        push_mse = jnp.sum(ds * ds) * inv_nn
        contrib += PULL_W * pull_mse + PUSH_W * jnp.maximum(MARGIN - push_mse, 0.0)
    acc_ref[0] += contrib

    @pl.when(i == nsteps - 1)
    def _fin():
        out_ref[0, 0] = acc_ref[0]


def _tc_sum(adj, ref_pull, ref_push):
    B, N, _ = adj.shape
    b_tc = B - _B_SC
    off = _B_SC // _G_TC
    nsteps = b_tc // _G_TC
    out = pl.pallas_call(
        functools.partial(_tc_body, g=_G_TC, n=N, nsteps=nsteps),
        grid=(nsteps,),
        in_specs=[
            pl.BlockSpec((_G_TC, N, N), lambda i: (i + off, 0, 0)),
            pl.BlockSpec((1, _G_TC, N, N), lambda i: (0, i + off, 0, 0)),
            pl.BlockSpec((1, _G_TC, N, N), lambda i: (0, i + off, 0, 0)),
        ],
        out_specs=pl.BlockSpec((1, 1), lambda i: (0, 0), memory_space=pltpu.SMEM),
        out_shape=jax.ShapeDtypeStruct((1, 1), jnp.float32),
        scratch_shapes=[pltpu.SMEM((1,), jnp.float32)],
    )(adj, ref_pull, ref_push)
    return out[0, 0]


def kernel(adj, ref_pull, ref_push):
    B, N, _ = adj.shape
    sc = _sc_contribs(adj, ref_pull, ref_push)
    tc = _tc_sum(adj, ref_pull, ref_push)
    return (tc + jnp.sum(sc.reshape(_NW, _L)[:, 0])) * (WEIGHT / B)


# speedup vs baseline: 4.2467x; 1.6283x over previous
import functools

import jax
import jax.numpy as jnp
from jax.experimental import pallas as pl
from jax.experimental.pallas import tpu as pltpu

THRESH = 0.5
MARGIN = 1.0
WEIGHT = 1.0
PULL_W = 1.0
PUSH_W = 1.0


def _loss_body(adj_lo, adj_hi, pull_lo, pull_hi, push_lo, push_hi,
               out_ref, acc_ref, *, g, n, b):
    i = pl.program_id(0)
    inv_nn = 1.0 / (n * n)

    @pl.when(i == 0)
    def _init():
        acc_ref[0] = 0.0

    contrib = 0.0
    for gg in range(g):
        a_lo = (adj_lo[gg] > THRESH).astype(jnp.float32)
        a_hi = (adj_hi[gg] > THRESH).astype(jnp.float32)
        dpl = a_lo - pull_lo[0, gg]
        dph = a_hi - pull_hi[0, gg]
        dsl = a_lo - push_lo[0, gg]
        dsh = a_hi - push_hi[0, gg]
        pull_mse = (jnp.sum(dpl * dpl) + jnp.sum(dph * dph)) * inv_nn
        push_mse = (jnp.sum(dsl * dsl) + jnp.sum(dsh * dsh)) * inv_nn
        contrib += PULL_W * pull_mse + PUSH_W * jnp.maximum(MARGIN - push_mse, 0.0)
    acc_ref[0] += WEIGHT * contrib

    @pl.when(i == (b // g) - 1)
    def _fin():
        out_ref[0, 0] = acc_ref[0] * (1.0 / b)


def kernel(adj, ref_pull, ref_push):
    B, N, _ = adj.shape
    G = 16
    H = N // 2
    grid = (B // G,)
    adj_spec = lambda rb: pl.BlockSpec((G, H, N), lambda i, rb=rb: (i, rb, 0))
    ref_spec = lambda rb: pl.BlockSpec((1, G, H, N), lambda i, rb=rb: (0, i, rb, 0))
    out = pl.pallas_call(
        functools.partial(_loss_body, g=G, n=N, b=B),
        grid=grid,
        in_specs=[adj_spec(0), adj_spec(1), ref_spec(0), ref_spec(1),
                  ref_spec(0), ref_spec(1)],
        out_specs=pl.BlockSpec((1, 1), lambda i: (0, 0), memory_space=pltpu.SMEM),
        out_shape=jax.ShapeDtypeStruct((1, 1), jnp.float32),
        scratch_shapes=[pltpu.SMEM((1,), jnp.float32)],
    )(adj, adj, ref_pull, ref_pull, ref_push, ref_push)
    return out[0, 0]


# speedup vs baseline: 4.2561x; 1.0022x over previous
import functools

import jax
import jax.numpy as jnp
from jax.experimental import pallas as pl
from jax.experimental.pallas import tpu as pltpu

THRESH = 0.5
MARGIN = 1.0
WEIGHT = 1.0
PULL_W = 1.0
PUSH_W = 1.0


def _loss_body(adj_ref, pull_ref, push_ref, out_ref, acc_ref, *, g, n, b):
    i = pl.program_id(0)
    inv_nn = 1.0 / (n * n)

    @pl.when(i == 0)
    def _init():
        acc_ref[0] = 0.0

    contrib = 0.0
    for gg in range(g):
        a = (adj_ref[gg] > THRESH).astype(jnp.float32)
        dp = a - pull_ref[0, gg]
        ds = a - push_ref[0, gg]
        pull_mse = jnp.sum(dp * dp) * inv_nn
        push_mse = jnp.sum(ds * ds) * inv_nn
        contrib += PULL_W * pull_mse + PUSH_W * jnp.maximum(MARGIN - push_mse, 0.0)
    acc_ref[0] += WEIGHT * contrib

    @pl.when(i == (b // g) - 1)
    def _fin():
        out_ref[0, 0] = acc_ref[0] * (1.0 / b)


def kernel(adj, ref_pull, ref_push):
    B, N, _ = adj.shape
    G = 16
    grid = (B // G,)
    out = pl.pallas_call(
        functools.partial(_loss_body, g=G, n=N, b=B),
        grid=grid,
        in_specs=[
            pl.BlockSpec((G, N, N), lambda i: (i, 0, 0)),
            pl.BlockSpec((1, G, N, N), lambda i: (0, i, 0, 0)),
            pl.BlockSpec((1, G, N, N), lambda i: (0, i, 0, 0)),
        ],
        out_specs=pl.BlockSpec((1, 1), lambda i: (0, 0), memory_space=pltpu.SMEM),
        out_shape=jax.ShapeDtypeStruct((1, 1), jnp.float32),
        scratch_shapes=[pltpu.SMEM((1,), jnp.float32)],
    )(adj, ref_pull, ref_push)
    return out[0, 0]
